# baseline (speedup 1.0000x reference)
import jax
import jax.numpy as jnp
from jax import lax
from jax.experimental import pallas as pl
from jax.experimental.pallas import tpu as pltpu
from jax.experimental.pallas import tpu_sc as plsc

NC = 2
NS = 16
L = 16
NW = NC * NS

BATCH = 4096
CELLS = 4096
BINS = 128
BLK = BATCH // NW
ROW = 2 * BLK
CCH = 128
NCHUNK = CELLS // CCH
NBUF = 3


def _histogram_body(gv_hbm, out_hbm, bufs, hist, sems):
    wid = lax.axis_index("s") * NC + lax.axis_index("c")

    lane = lax.iota(jnp.int32, L)
    ones = jnp.ones((L,), jnp.int32)
    zeros = jnp.zeros((L,), jnp.int32)
    rows = [lane + p * L for p in range(BLK // L)]

    for s in range(BLK):
        for k in range(BINS // L):
            hist[s, pl.ds(k * L, L)] = zeros

    bufv = [b.reshape(CCH, 2, BLK) for b in bufs]

    for j in range(NBUF):
        pltpu.async_copy(
            gv_hbm.at[pl.ds(j * CCH, CCH), wid, :, :], bufv[j], sems.at[j]
        )

    def scatter_chunk(buf):
        @plsc.parallel_loop(0, CCH, unroll=2)
        def _(j):
            jt = j * 2
            for p in range(BLK // L):
                t = buf[jt, pl.ds(p * L, L)]
                c = buf[jt + 1, pl.ds(p * L, L)]
                ids = lax.shift_left(t, 3) + c
                plsc.addupdate_scatter(hist, [rows[p], ids], ones)

    def outer(g, _):
        for j in range(NBUF):
            chunk = g * NBUF + j
            c0 = chunk * CCH
            pltpu.make_async_copy(
                gv_hbm.at[pl.ds(c0, CCH), wid, :, :], bufv[j], sems.at[j]
            ).wait()

            scatter_chunk(bufs[j])

            @pl.when(chunk + NBUF < NCHUNK)
            def _():
                pltpu.async_copy(
                    gv_hbm.at[pl.ds(c0 + NBUF * CCH, CCH), wid, :, :],
                    bufv[j],
                    sems.at[j],
                )
        return 0

    lax.fori_loop(0, NCHUNK // NBUF, outer, 0)
    for j in range(NCHUNK % NBUF):
        c0 = (NCHUNK - NCHUNK % NBUF + j) * CCH
        pltpu.make_async_copy(
            gv_hbm.at[pl.ds(c0, CCH), wid, :, :], bufv[j], sems.at[j]
        ).wait()
        scatter_chunk(bufs[j])

    pltpu.sync_copy(hist, out_hbm.at[pl.ds(wid * BLK, BLK), :])


def _sc_histogram(gv):
    mesh = plsc.VectorSubcoreMesh(
        core_axis_name="c", subcore_axis_name="s", num_cores=NC,
        num_subcores=NS,
    )

    def body(gv_hbm, out_hbm, b0, b1, b2, hist, sems):
        _histogram_body(gv_hbm, out_hbm, (b0, b1, b2), hist, sems)

    return pl.kernel(
        body,
        out_type=jax.ShapeDtypeStruct((BATCH, BINS), jnp.int32),
        mesh=mesh,
        compiler_params=pltpu.CompilerParams(
            needs_layout_passes=False, skip_device_barrier=True
        ),
        scratch_types=[
            pltpu.VMEM((2 * CCH, BLK), jnp.int32),
            pltpu.VMEM((2 * CCH, BLK), jnp.int32),
            pltpu.VMEM((2 * CCH, BLK), jnp.int32),
            pltpu.VMEM((BLK, BINS), jnp.int32),
            pltpu.SemaphoreType.DMA((NBUF,)),
        ],
    )(gv)


@jax.jit
def kernel(grid_state):
    g2 = jnp.transpose(grid_state, (1, 2, 3, 0))
    g3 = g2.reshape(64, 64, 2, NW, BLK)
    g4 = jnp.transpose(g3, (0, 1, 3, 2, 4))
    gv = g4.reshape(CELLS, NW, 2, BLK)
    return _sc_histogram(gv)


# speedup vs baseline: 1.0240x; 1.0240x over previous
import jax
import jax.numpy as jnp
from jax import lax
from jax.experimental import pallas as pl
from jax.experimental.pallas import tpu as pltpu
from jax.experimental.pallas import tpu_sc as plsc

NC = 2
NS = 16
L = 16
NW = NC * NS

BATCH = 4096
CELLS = 4096
BINS = 128
BLK = BATCH // NW
ROW = 2 * BLK
CCH = 128
NCHUNK = CELLS // CCH
NBUF = 2


def _histogram_body(gv_hbm, out_hbm, bufs, hist, sems):
    wid = lax.axis_index("s") * NC + lax.axis_index("c")

    lane = lax.iota(jnp.int32, L)
    ones = jnp.ones((L,), jnp.int32)
    zeros = jnp.zeros((L,), jnp.int32)
    rows = [lane + p * L for p in range(BLK // L)]

    for s in range(BLK):
        for k in range(BINS // L):
            hist[s, pl.ds(k * L, L)] = zeros

    bufv = [b.reshape(CCH, 2, BLK) for b in bufs]

    for j in range(NBUF):
        pltpu.async_copy(
            gv_hbm.at[pl.ds(j * CCH, CCH), wid, :, :], bufv[j], sems.at[j]
        )

    def scatter_chunk(buf):
        @plsc.parallel_loop(0, CCH, unroll=2)
        def _(j):
            jt = j * 2
            for p in range(BLK // L):
                t = buf[jt, pl.ds(p * L, L)]
                c = buf[jt + 1, pl.ds(p * L, L)]
                ids = lax.shift_left(t, 3) + c
                plsc.addupdate_scatter(hist, [rows[p], ids], ones)

    def outer(g, _):
        for j in range(NBUF):
            chunk = g * NBUF + j
            c0 = chunk * CCH
            pltpu.make_async_copy(
                gv_hbm.at[pl.ds(c0, CCH), wid, :, :], bufv[j], sems.at[j]
            ).wait()

            scatter_chunk(bufs[j])

            @pl.when(chunk + NBUF < NCHUNK)
            def _():
                pltpu.async_copy(
                    gv_hbm.at[pl.ds(c0 + NBUF * CCH, CCH), wid, :, :],
                    bufv[j],
                    sems.at[j],
                )
        return 0

    lax.fori_loop(0, NCHUNK // NBUF, outer, 0)
    for j in range(NCHUNK % NBUF):
        c0 = (NCHUNK - NCHUNK % NBUF + j) * CCH
        pltpu.make_async_copy(
            gv_hbm.at[pl.ds(c0, CCH), wid, :, :], bufv[j], sems.at[j]
        ).wait()
        scatter_chunk(bufs[j])

    pltpu.sync_copy(hist, out_hbm.at[pl.ds(wid * BLK, BLK), :])


def _sc_histogram(gv):
    mesh = plsc.VectorSubcoreMesh(
        core_axis_name="c", subcore_axis_name="s", num_cores=NC,
        num_subcores=NS,
    )

    def body(gv_hbm, out_hbm, b0, b1, hist, sems):
        _histogram_body(gv_hbm, out_hbm, (b0, b1), hist, sems)

    return pl.kernel(
        body,
        out_type=jax.ShapeDtypeStruct((BATCH, BINS), jnp.int32),
        mesh=mesh,
        compiler_params=pltpu.CompilerParams(needs_layout_passes=False),
        scratch_types=[
            pltpu.VMEM((2 * CCH, BLK), jnp.int32),
            pltpu.VMEM((2 * CCH, BLK), jnp.int32),
            pltpu.VMEM((BLK, BINS), jnp.int32),
            pltpu.SemaphoreType.DMA((NBUF,)),
        ],
    )(gv)


@jax.jit
def kernel(grid_state):
    g2 = jnp.transpose(grid_state, (1, 2, 3, 0))
    g3 = g2.reshape(64, 64, 2, NW, BLK)
    g4 = jnp.transpose(g3, (0, 1, 3, 2, 4))
    gv = g4.reshape(CELLS, NW, 2, BLK)
    return _sc_histogram(gv)


# speedup vs baseline: 1.0248x; 1.0008x over previous
import jax
import jax.numpy as jnp
from jax import lax
from jax.experimental import pallas as pl
from jax.experimental.pallas import tpu as pltpu
from jax.experimental.pallas import tpu_sc as plsc

NC = 2
NS = 16
L = 16
NW = NC * NS

BATCH = 4096
CELLS = 4096
BINS = 128
BLK = BATCH // NW
CCH = 128
NCHUNK = CELLS // CCH
NBUF = 2


def _histogram_body(gv_hbm, out_hbm, bufs, hist, sems):
    wid = lax.axis_index("s") * NC + lax.axis_index("c")

    lane = lax.iota(jnp.int32, L)
    ones = jnp.ones((L,), jnp.int32)
    zeros = jnp.zeros((L,), jnp.int32)
    rows = [lane + p * L for p in range(BLK // L)]

    for s in range(BLK):
        for k in range(BINS // L):
            hist[s, pl.ds(k * L, L)] = zeros

    bufv = [b.reshape(CCH, 2, BLK) for b in bufs]

    for j in range(NBUF):
        pltpu.async_copy(
            gv_hbm.at[pl.ds(j * CCH, CCH), wid, :, :], bufv[j], sems.at[j]
        )

    def scatter_chunk(buf):
        @plsc.parallel_loop(0, CCH, unroll=2)
        def _(j):
            jt = j * 2
            for p in range(BLK // L):
                t = buf[jt, pl.ds(p * L, L)]
                c = buf[jt + 1, pl.ds(p * L, L)]
                ids = lax.shift_left(t, 3) + c
                plsc.addupdate_scatter(hist, [rows[p], ids], ones)

    def outer(g, _):
        for j in range(NBUF):
            chunk = g * NBUF + j
            c0 = chunk * CCH
            pltpu.make_async_copy(
                gv_hbm.at[pl.ds(c0, CCH), wid, :, :], bufv[j], sems.at[j]
            ).wait()

            scatter_chunk(bufs[j])

            @pl.when(chunk + NBUF < NCHUNK)
            def _():
                pltpu.async_copy(
                    gv_hbm.at[pl.ds(c0 + NBUF * CCH, CCH), wid, :, :],
                    bufv[j],
                    sems.at[j],
                )
        return 0

    lax.fori_loop(0, NCHUNK // NBUF, outer, 0)
    for j in range(NCHUNK % NBUF):
        c0 = (NCHUNK - NCHUNK % NBUF + j) * CCH
        pltpu.make_async_copy(
            gv_hbm.at[pl.ds(c0, CCH), wid, :, :], bufv[j], sems.at[j]
        ).wait()
        scatter_chunk(bufs[j])

    pltpu.sync_copy(hist, out_hbm.at[pl.ds(wid * BLK, BLK), :])


def _sc_histogram(gv):
    mesh = plsc.VectorSubcoreMesh(
        core_axis_name="c", subcore_axis_name="s", num_cores=NC,
        num_subcores=NS,
    )

    def body(gv_hbm, out_hbm, b0, b1, hist, sems):
        _histogram_body(gv_hbm, out_hbm, (b0, b1), hist, sems)

    return pl.kernel(
        body,
        out_type=jax.ShapeDtypeStruct((BATCH, BINS), jnp.int32),
        mesh=mesh,
        compiler_params=pltpu.CompilerParams(needs_layout_passes=False),
        scratch_types=[
            pltpu.VMEM((2 * CCH, BLK), jnp.int32),
            pltpu.VMEM((2 * CCH, BLK), jnp.int32),
            pltpu.VMEM((BLK, BINS), jnp.int32),
            pltpu.SemaphoreType.DMA((NBUF,)),
        ],
    )(gv)


@jax.jit
def kernel(grid_state):
    g2 = jnp.transpose(grid_state, (1, 2, 3, 0))
    g3 = g2.reshape(64, 64, 2, NW, BLK)
    g4 = jnp.transpose(g3, (0, 1, 3, 2, 4))
    gv = g4.reshape(CELLS, NW, 2, BLK)
    return _sc_histogram(gv)
